# baseline (speedup 1.0000x reference)
import jax
import jax.numpy as jnp
from jax import lax
from jax.experimental import pallas as pl
from jax.experimental.pallas import tpu as pltpu
from jax.experimental.pallas import tpu_sc as plsc

_B = 4
_D = 256
_H = 32
_W = 32
_L = 16


def _pos_body(row_hbm, col_hbm, out_hbm, slab, buf, sem):
    cid = lax.axis_index("c")
    sid = lax.axis_index("s")
    lane = lax.iota(jnp.int32, _L)

    @pl.when(cid == 0)
    def _():
        pltpu.sync_copy(col_hbm.at[pl.ds(0, _W)], slab)
        for i in range(_L):
            ci = jnp.full((_L,), i, jnp.int32) + sid * _L
            va = plsc.load_gather(slab, [lane, ci])
            vb = plsc.load_gather(slab, [lane + _L, ci])
            for y in range(_H):
                buf[i, y, pl.ds(0, _L)] = va
                buf[i, y, pl.ds(_L, _L)] = vb

    @pl.when(cid == 1)
    def _():
        pltpu.sync_copy(row_hbm.at[pl.ds(0, _H)], slab)
        for i in range(_L):
            ci = jnp.full((_L,), i, jnp.int32) + sid * _L
            for y in range(_H):
                v = plsc.load_gather(slab, [jnp.full((_L,), y, jnp.int32), ci])
                buf[i, y, pl.ds(0, _L)] = v
                buf[i, y, pl.ds(_L, _L)] = v

    ch = cid * _D + sid * _L
    copies = [
        pltpu.async_copy(buf, out_hbm.at[b, pl.ds(ch, _L)], sem)
        for b in range(_B)
    ]
    for cp in copies:
        cp.wait()


def kernel(img, mask, row_embed, col_embed):
    del img, mask
    mesh = plsc.VectorSubcoreMesh(core_axis_name="c", subcore_axis_name="s")
    fn = pl.kernel(
        _pos_body,
        mesh=mesh,
        out_type=jax.ShapeDtypeStruct((_B, 2 * _D, _H, _W), jnp.float32),
        scratch_types=[
            pltpu.VMEM((_W, _D), jnp.float32),
            pltpu.VMEM((_L, _H, _W), jnp.float32),
            pltpu.SemaphoreType.DMA,
        ],
        compiler_params=pltpu.CompilerParams(
            use_tc_tiling_on_sc=False, needs_layout_passes=False
        ),
    )
    return fn(row_embed, col_embed)


# speedup vs baseline: 1.4808x; 1.4808x over previous
import jax
import jax.numpy as jnp
from jax import lax
from jax.experimental import pallas as pl
from jax.experimental.pallas import tpu as pltpu
from jax.experimental.pallas import tpu_sc as plsc

_B = 4
_D = 256
_H = 32
_W = 32
_L = 16


def _pos_body(row_hbm, col_hbm, out_hbm, slab, buf, sem):
    cid = lax.axis_index("c")
    sid = lax.axis_index("s")
    lane = lax.iota(jnp.int32, _L)

    @pl.when(cid == 0)
    def _():
        pltpu.sync_copy(col_hbm.at[pl.ds(0, _W)], slab)
        for i in range(_L):
            ci = jnp.full((_L,), i, jnp.int32) + sid * _L
            va = plsc.load_gather(slab, [lane, ci])
            vb = plsc.load_gather(slab, [lane + _L, ci])
            for y in range(_H):
                buf[i, pl.ds(y * _W, _L)] = va
                buf[i, pl.ds(y * _W + _L, _L)] = vb

    @pl.when(cid == 1)
    def _():
        pltpu.sync_copy(row_hbm.at[pl.ds(0, _H)], slab)
        for i in range(_L):
            ci = jnp.full((_L,), i, jnp.int32) + sid * _L
            w0 = plsc.load_gather(slab, [lane, ci])
            w1 = plsc.load_gather(slab, [lane + _L, ci])
            for y in range(_H):
                src = w0 if y < _L else w1
                v = jnp.take(src, jnp.full((_L,), y % _L, jnp.int32))
                buf[i, pl.ds(y * _W, _L)] = v
                buf[i, pl.ds(y * _W + _L, _L)] = v

    ch = cid * _D + sid * _L
    copies = [
        pltpu.async_copy(buf, out_hbm.at[b, pl.ds(ch, _L)], sem)
        for b in range(_B)
    ]
    for cp in copies:
        cp.wait()


def kernel(img, mask, row_embed, col_embed):
    del img, mask
    mesh = plsc.VectorSubcoreMesh(core_axis_name="c", subcore_axis_name="s")
    fn = pl.kernel(
        _pos_body,
        mesh=mesh,
        out_type=jax.ShapeDtypeStruct((_B, 2 * _D, _H * _W), jnp.float32),
        scratch_types=[
            pltpu.VMEM((_W, _D), jnp.float32),
            pltpu.VMEM((_L, _H * _W), jnp.float32),
            pltpu.SemaphoreType.DMA,
        ],
        compiler_params=pltpu.CompilerParams(
            use_tc_tiling_on_sc=False, needs_layout_passes=False
        ),
    )
    out = fn(row_embed, col_embed)
    return out.reshape(_B, 2 * _D, _H, _W)


# speedup vs baseline: 2.7350x; 1.8469x over previous
import jax
import jax.numpy as jnp
from jax import lax
from jax.experimental import pallas as pl
from jax.experimental.pallas import tpu as pltpu
from jax.experimental.pallas import tpu_sc as plsc

_B = 4
_D = 256
_H = 32
_W = 32
_L = 16


def _pos_body(row_hbm, col_hbm, out_hbm, idx, buf, sem):
    cid = lax.axis_index("c")
    sid = lax.axis_index("s")
    y = cid * (_H // 2) + sid
    yv = jnp.full((_L,), y, jnp.int32)
    idx[pl.ds(0, _L)] = yv
    idx[pl.ds(_L, _L)] = yv

    cp_col = pltpu.async_copy(
        col_hbm.at[pl.ds(0, _W)], buf.at[:, pl.ds(0, _D)], sem
    )
    cp_row = pltpu.async_copy(
        row_hbm.at[idx], buf.at[:, pl.ds(_D, _D)], sem
    )
    cp_col.wait()
    cp_row.wait()

    copies = [
        pltpu.async_copy(buf, out_hbm.at[b, y], sem) for b in range(_B)
    ]
    for cp in copies:
        cp.wait()


def kernel(img, mask, row_embed, col_embed):
    del img, mask
    mesh = plsc.VectorSubcoreMesh(core_axis_name="c", subcore_axis_name="s")
    fn = pl.kernel(
        _pos_body,
        mesh=mesh,
        out_type=jax.ShapeDtypeStruct((_B, _H, _W, 2 * _D), jnp.float32),
        scratch_types=[
            pltpu.VMEM((_W,), jnp.int32),
            pltpu.VMEM((_W, 2 * _D), jnp.float32),
            pltpu.SemaphoreType.DMA,
        ],
        compiler_params=pltpu.CompilerParams(use_tc_tiling_on_sc=True),
    )
    out_t = fn(row_embed, col_embed)
    return jnp.transpose(out_t, (0, 3, 1, 2))
